# baseline (speedup 1.0000x reference)
import jax
import jax.numpy as jnp
from jax.experimental import pallas as pl
from jax.experimental.pallas import tpu as pltpu

_ALPHA = 0.5
_BB = 8


def _align_body(x_ref, p_ref, p2_ref, o_ref):
    xb = x_ref[...]
    hw = xb.shape[1]
    feat = jnp.sum(xb, axis=1) * (1.0 / hw)
    f2 = jnp.sum(feat * feat, axis=1, keepdims=True)
    feat_bf = feat.astype(jnp.bfloat16)
    k = p_ref.shape[0]
    hk = k // 2
    nearest = jnp.zeros(feat.shape, jnp.float32)
    d2h, iih = [], []
    for h in range(2):
        ph = p_ref[h * hk:(h + 1) * hk, :]
        dots = jax.lax.dot_general(
            feat_bf, ph, (((1,), (1,)), ((), ())),
            preferred_element_type=jnp.float32)
        p2h = p2_ref[:, h * hk:(h + 1) * hk]
        d2h.append(jnp.maximum((f2 + p2h) - 2.0 * dots, 0.0))
        iih.append(jax.lax.broadcasted_iota(jnp.int32, d2h[0].shape, 1) + h * hk)
    m = jnp.minimum(jnp.min(d2h[0], axis=1, keepdims=True),
                    jnp.min(d2h[1], axis=1, keepdims=True))
    idx = jnp.minimum(
        jnp.min(jnp.where(d2h[0] <= m, iih[0], jnp.int32(k)),
                axis=1, keepdims=True),
        jnp.min(jnp.where(d2h[1] <= m, iih[1], jnp.int32(k)),
                axis=1, keepdims=True))
    for h in range(2):
        ph = p_ref[h * hk:(h + 1) * hk, :]
        onehot = (iih[h] == idx).astype(jnp.bfloat16)
        nearest = nearest + jax.lax.dot_general(
            onehot, ph, (((1,), (0,)), ((), ())),
            preferred_element_type=jnp.float32)
    delta = _ALPHA * (nearest - feat)
    o_ref[...] = xb + delta[:, None, :]


def kernel(x, prototypes):
    B, C, H, W = x.shape
    K = prototypes.shape[0]
    HW = H * W
    xt = x.transpose(0, 2, 3, 1).reshape(B, HW, C)
    p_bf = prototypes.astype(jnp.bfloat16)
    p2 = jnp.sum(prototypes * prototypes, axis=1)[None, :]
    out_t = pl.pallas_call(
        _align_body,
        grid=(B // _BB,),
        in_specs=[
            pl.BlockSpec((_BB, HW, C), lambda i: (i, 0, 0)),
            pl.BlockSpec((K, C), lambda i: (0, 0)),
            pl.BlockSpec((1, K), lambda i: (0, 0)),
        ],
        out_specs=pl.BlockSpec((_BB, HW, C), lambda i: (i, 0, 0)),
        out_shape=jax.ShapeDtypeStruct((B, HW, C), x.dtype),
        compiler_params=pltpu.CompilerParams(
            dimension_semantics=("parallel",)),
    )(xt, p_bf, p2)
    return out_t.reshape(B, H, W, C).transpose(0, 3, 1, 2)
